# speedup vs baseline: 10.7266x; 10.7266x over previous
import functools

import jax
import jax.numpy as jnp
from jax import lax
from jax.experimental import pallas as pl
from jax.experimental.pallas import tpu as pltpu
from jax.experimental.pallas import tpu_sc as plsc

C, H, W = 19, 512, 512
HW = H * W
K = 100
KPAD = 112
D = 128
NC, NS = 2, 16
THRES = 0.1
NEG = -1.0


def _scores_topk_body(x_ref, svals_ref, scats_ref, ssp_ref,
                      score_ref, bm_ref, rawv_ref, rawi_ref):
    c = pl.program_id(0)
    x = x_ref[0, 0]

    s = 1.0 / (1.0 + jnp.exp(-x))

    zcol = jnp.zeros((H, 1), jnp.float32)
    zrow = jnp.zeros((1, W), jnp.float32)
    rs = s + jnp.concatenate([s[:, 1:], zcol], axis=1) \
           + jnp.concatenate([zcol, s[:, :-1]], axis=1)
    sum9 = rs + jnp.concatenate([rs[1:, :], zrow], axis=0) \
              + jnp.concatenate([zrow, rs[:-1, :]], axis=0)
    cent = (s + sum9 * (1.0 / 9.0)) * 0.5

    ninf = jnp.float32(-jnp.inf)
    icol = jnp.full((H, 1), ninf)
    irow = jnp.full((1, W), ninf)
    mw = jnp.maximum(cent,
                     jnp.maximum(jnp.concatenate([cent[:, 1:], icol], axis=1),
                                 jnp.concatenate([icol, cent[:, :-1]], axis=1)))
    m3 = jnp.maximum(mw,
                     jnp.maximum(jnp.concatenate([mw[1:, :], irow], axis=0),
                                 jnp.concatenate([irow, mw[:-1, :]], axis=0)))
    score = jnp.where(m3 == cent, cent, 0.0)

    r0 = pl.multiple_of(c * H, H)
    score_ref[pl.ds(r0, H), :] = score

    s1 = jnp.max(score.reshape(H // 8, 8, W), axis=1)
    bm_c = jnp.stack(
        [jnp.max(s1[:, j * 128:(j + 1) * 128], axis=1) for j in range(4)],
        axis=1)
    bm_ref[pl.ds(c, 1)] = bm_c.reshape(1, H // 8, 4)

    @pl.when(c == C - 1)
    def _extract():
        lane = lax.broadcasted_iota(jnp.int32, (1, 128), 1)
        rawv_ref[...] = jnp.zeros((1, 128), jnp.float32)
        rawi_ref[...] = jnp.zeros((1, 128), jnp.int32)

        ic = lax.broadcasted_iota(jnp.int32, (C, H // 8, 4), 0)
        irb = lax.broadcasted_iota(jnp.int32, (C, H // 8, 4), 1)
        icb = lax.broadcasted_iota(jnp.int32, (C, H // 8, 4), 2)
        bflat = (ic * (H // 8) + irb) * 4 + icb
        ib = (lax.broadcasted_iota(jnp.int32, (8, 128), 0) * 128
              + lax.broadcasted_iota(jnp.int32, (8, 128), 1))

        def body(i, carry):
            bm = bm_ref[...]
            m = jnp.max(bm)
            bsel = jnp.min(jnp.where(bm == m, bflat, jnp.int32(1 << 30)))
            tr0 = pl.multiple_of((bsel // 4) * 8, 8)
            tc0 = pl.multiple_of((bsel % 4) * 128, 128)
            blk = score_ref[pl.ds(tr0, 8), pl.ds(tc0, 128)]
            v = jnp.max(blk)
            pos = jnp.min(jnp.where(blk == v, ib, jnp.int32(1 << 30)))
            gidx = (tr0 + pos // 128) * W + (tc0 + pos % 128)
            blk_new = jnp.where(ib == pos, NEG, blk)
            score_ref[pl.ds(tr0, 8), pl.ds(tc0, 128)] = blk_new
            bm_ref[...] = jnp.where(bflat == bsel, jnp.max(blk_new), bm)
            rawv_ref[...] = jnp.where(lane == i, v, rawv_ref[...])
            rawi_ref[...] = jnp.where(lane == i, gidx, rawi_ref[...])
            return carry

        lax.fori_loop(0, K, body, 0)

        vals = rawv_ref[...]
        idx = rawi_ref[...]
        valid = lane < K
        keep = jnp.logical_and(vals > THRES, valid)
        svals_ref[...] = jnp.where(keep, vals, 0.0)
        scats_ref[...] = jnp.where(keep, idx // HW, 0)
        ssp_ref[...] = jnp.where(valid, idx % HW, 0)


def _detect(thing_map):
    return pl.pallas_call(
        _scores_topk_body,
        grid=(C,),
        in_specs=[pl.BlockSpec((1, 1, H, W), lambda c: (0, c, 0, 0))],
        out_specs=[
            pl.BlockSpec((1, 128), lambda c: (0, 0)),
            pl.BlockSpec((1, 128), lambda c: (0, 0)),
            pl.BlockSpec((1, 128), lambda c: (0, 0)),
        ],
        out_shape=[
            jax.ShapeDtypeStruct((1, 128), jnp.float32),
            jax.ShapeDtypeStruct((1, 128), jnp.int32),
            jax.ShapeDtypeStruct((1, 128), jnp.int32),
        ],
        scratch_shapes=[
            pltpu.VMEM((C * H, W), jnp.float32),
            pltpu.VMEM((C, H // 8, 4), jnp.float32),
            pltpu.VMEM((1, 128), jnp.float32),
            pltpu.VMEM((1, 128), jnp.int32),
        ],
        compiler_params=pltpu.CompilerParams(
            dimension_semantics=("arbitrary",)),
    )(thing_map)


def _sc_gather_body(kflat_hbm, sp_hbm, out_hbm, sp_v, addr_v, row_v, sem):
    w = lax.axis_index("s") * NC + lax.axis_index("c")
    pltpu.sync_copy(sp_hbm.at[pl.ds(0, KPAD)], sp_v)
    for j in range(D // (NC * NS)):
        d = w * (D // (NC * NS)) + j
        base = d * HW
        for t in range(KPAD // 16):
            sl = pl.ds(t * 16, 16)
            addr_v[sl] = sp_v[sl] + base
        pltpu.async_copy(kflat_hbm.at[addr_v], row_v, sem).wait()
        pltpu.sync_copy(row_v, out_hbm.at[d])


def _sc_gather(kflat, sp):
    mesh = plsc.VectorSubcoreMesh(core_axis_name="c", subcore_axis_name="s")
    f = functools.partial(
        pl.kernel,
        mesh=mesh,
        out_type=jax.ShapeDtypeStruct((D, KPAD), jnp.float32),
        scratch_types=[
            pltpu.VMEM((KPAD,), jnp.int32),
            pltpu.VMEM((KPAD,), jnp.int32),
            pltpu.VMEM((KPAD,), jnp.float32),
            pltpu.SemaphoreType.DMA,
        ],
    )(_sc_gather_body)
    return f(kflat, sp)


def kernel(thing_map, kernel_space):
    svals, scats, ssp = _detect(thing_map)
    kflat = kernel_space.reshape(D * HW)
    outT = _sc_gather(kflat, ssp.reshape(128))
    kernels = jnp.transpose(outT[:, :K])[None]
    scores = svals[:, :K]
    cats = scats[:, :K]
    return kernels, cats, scores


# speedup vs baseline: 12.4351x; 1.1593x over previous
import functools

import jax
import jax.numpy as jnp
from jax import lax
from jax.experimental import pallas as pl
from jax.experimental.pallas import tpu as pltpu
from jax.experimental.pallas import tpu_sc as plsc

C, H, W = 19, 512, 512
HW = H * W
K = 100
KPAD = 112
D = 128
NC, NS = 2, 16
THRES = 0.1
NEG = -1.0


def _scores_topk_body(x_ref, svals_ref, scats_ref, ssp_ref,
                      score_ref, bm_ref, rawv_ref, rawi_ref):
    c = pl.program_id(0)
    x = x_ref[0, 0]

    s = 1.0 / (1.0 + jnp.exp(-x))

    zcol = jnp.zeros((H, 1), jnp.float32)
    zrow = jnp.zeros((1, W), jnp.float32)
    rs = s + jnp.concatenate([s[:, 1:], zcol], axis=1) \
           + jnp.concatenate([zcol, s[:, :-1]], axis=1)
    sum9 = rs + jnp.concatenate([rs[1:, :], zrow], axis=0) \
              + jnp.concatenate([zrow, rs[:-1, :]], axis=0)
    cent = (s + sum9 * (1.0 / 9.0)) * 0.5

    ninf = jnp.float32(-jnp.inf)
    icol = jnp.full((H, 1), ninf)
    irow = jnp.full((1, W), ninf)
    mw = jnp.maximum(cent,
                     jnp.maximum(jnp.concatenate([cent[:, 1:], icol], axis=1),
                                 jnp.concatenate([icol, cent[:, :-1]], axis=1)))
    m3 = jnp.maximum(mw,
                     jnp.maximum(jnp.concatenate([mw[1:, :], irow], axis=0),
                                 jnp.concatenate([irow, mw[:-1, :]], axis=0)))
    score = jnp.where(m3 == cent, cent, 0.0)

    r0 = pl.multiple_of(c * H, H)
    score_ref[pl.ds(r0, H), :] = score

    s1 = jnp.max(score.reshape(H // 8, 8, W), axis=1)
    bm_c = jnp.max(s1, axis=1)
    bm_ref[pl.ds(c, 1)] = bm_c.reshape(1, H // 8)

    @pl.when(c == C - 1)
    def _extract():
        lane = lax.broadcasted_iota(jnp.int32, (1, 128), 1)
        rawv_ref[...] = jnp.zeros((1, 128), jnp.float32)
        rawi_ref[...] = jnp.zeros((1, 128), jnp.int32)

        bflat = (lax.broadcasted_iota(jnp.int32, (C, H // 8), 0) * (H // 8)
                 + lax.broadcasted_iota(jnp.int32, (C, H // 8), 1))
        ib = (lax.broadcasted_iota(jnp.int32, (8, W), 0) * W
              + lax.broadcasted_iota(jnp.int32, (8, W), 1))

        def body(i, carry):
            bm = bm_ref[...]
            m = jnp.max(bm)
            bsel = jnp.min(jnp.where(bm == m, bflat, jnp.int32(1 << 30)))
            tr0 = pl.multiple_of(bsel * 8, 8)
            blk = score_ref[pl.ds(tr0, 8), :]
            v = jnp.max(blk)
            pos = jnp.min(jnp.where(blk == v, ib, jnp.int32(1 << 30)))
            gidx = tr0 * W + pos
            blk_new = jnp.where(ib == pos, NEG, blk)
            score_ref[pl.ds(tr0, 8), :] = blk_new
            bm_ref[...] = jnp.where(bflat == bsel, jnp.max(blk_new), bm)
            rawv_ref[...] = jnp.where(lane == i, v, rawv_ref[...])
            rawi_ref[...] = jnp.where(lane == i, gidx, rawi_ref[...])
            return carry

        lax.fori_loop(0, K, body, 0)

        vals = rawv_ref[...]
        idx = rawi_ref[...]
        valid = lane < K
        keep = jnp.logical_and(vals > THRES, valid)
        svals_ref[...] = jnp.where(keep, vals, 0.0)
        scats_ref[...] = jnp.where(keep, idx // HW, 0)
        sp = jnp.where(valid, idx % HW, 0)
        spb = jnp.transpose(jnp.broadcast_to(sp, (128, 128)))
        dmat = lax.broadcasted_iota(jnp.int32, (128, 128), 1) * HW
        ssp_ref[...] = spb + dmat


def _detect(thing_map):
    return pl.pallas_call(
        _scores_topk_body,
        grid=(C,),
        in_specs=[pl.BlockSpec((1, 1, H, W), lambda c: (0, c, 0, 0))],
        out_specs=[
            pl.BlockSpec((1, 128), lambda c: (0, 0)),
            pl.BlockSpec((1, 128), lambda c: (0, 0)),
            pl.BlockSpec((128, 128), lambda c: (0, 0)),
        ],
        out_shape=[
            jax.ShapeDtypeStruct((1, 128), jnp.float32),
            jax.ShapeDtypeStruct((1, 128), jnp.int32),
            jax.ShapeDtypeStruct((128, 128), jnp.int32),
        ],
        scratch_shapes=[
            pltpu.VMEM((C * H, W), jnp.float32),
            pltpu.VMEM((C, H // 8), jnp.float32),
            pltpu.VMEM((1, 128), jnp.float32),
            pltpu.VMEM((1, 128), jnp.int32),
        ],
        compiler_params=pltpu.CompilerParams(
            dimension_semantics=("arbitrary",)),
    )(thing_map)


def _sc_gather_body(kflat_hbm, addr_hbm, out_hbm, addr_v, row_v, sem):
    w = lax.axis_index("s") * NC + lax.axis_index("c")
    for j in range(128 // (NC * NS)):
        k = w * (128 // (NC * NS)) + j
        pltpu.sync_copy(addr_hbm.at[k], addr_v)
        pltpu.async_copy(kflat_hbm.at[addr_v], row_v, sem).wait()
        pltpu.sync_copy(row_v, out_hbm.at[k])


def _sc_gather(kflat, addr):
    mesh = plsc.VectorSubcoreMesh(core_axis_name="c", subcore_axis_name="s")
    f = functools.partial(
        pl.kernel,
        mesh=mesh,
        out_type=jax.ShapeDtypeStruct((128, D), jnp.float32),
        scratch_types=[
            pltpu.VMEM((D,), jnp.int32),
            pltpu.VMEM((D,), jnp.float32),
            pltpu.SemaphoreType.DMA,
        ],
    )(_sc_gather_body)
    return f(kflat, addr)


def kernel(thing_map, kernel_space):
    svals, scats, addr = _detect(thing_map)
    kflat = kernel_space.reshape(D * HW)
    rows = _sc_gather(kflat, addr)
    kernels = rows[:K][None]
    scores = svals[:, :K]
    cats = scats[:, :K]
    return kernels, cats, scores


# speedup vs baseline: 15.5007x; 1.2465x over previous
import functools

import jax
import jax.numpy as jnp
from jax import lax
from jax.experimental import pallas as pl
from jax.experimental.pallas import tpu as pltpu
from jax.experimental.pallas import tpu_sc as plsc

C, H, W = 19, 512, 512
HW = H * W
K = 100
KPAD = 112
D = 128
NC, NS = 2, 16
THRES = 0.1
NEG = -1.0


def _scores_topk_body(x_ref, ks_ref, svals_ref, scats_ref, kout_ref,
                      score_ref, bm_ref, rawv_ref, rawi_ref, rawp_ref,
                      stage_ref, dma_sem):
    c = pl.program_id(0)
    x = x_ref[0, 0]

    s = 1.0 / (1.0 + jnp.exp(-x))

    zcol = jnp.zeros((H, 1), jnp.float32)
    zrow = jnp.zeros((1, W), jnp.float32)
    rs = s + jnp.concatenate([s[:, 1:], zcol], axis=1) \
           + jnp.concatenate([zcol, s[:, :-1]], axis=1)
    sum9 = rs + jnp.concatenate([rs[1:, :], zrow], axis=0) \
              + jnp.concatenate([zrow, rs[:-1, :]], axis=0)
    cent = (s + sum9 * (1.0 / 9.0)) * 0.5

    ninf = jnp.float32(-jnp.inf)
    icol = jnp.full((H, 1), ninf)
    irow = jnp.full((1, W), ninf)
    mw = jnp.maximum(cent,
                     jnp.maximum(jnp.concatenate([cent[:, 1:], icol], axis=1),
                                 jnp.concatenate([icol, cent[:, :-1]], axis=1)))
    m3 = jnp.maximum(mw,
                     jnp.maximum(jnp.concatenate([mw[1:, :], irow], axis=0),
                                 jnp.concatenate([irow, mw[:-1, :]], axis=0)))
    score = jnp.where(m3 == cent, cent, 0.0)

    r0 = pl.multiple_of(c * H, H)
    score_ref[pl.ds(r0, H), :] = score

    s1 = jnp.max(score.reshape(H // 8, 8, W), axis=1)
    bm_c = jnp.max(s1, axis=1)
    bm_ref[pl.ds(c, 1)] = bm_c.reshape(1, H // 8)

    @pl.when(c == C - 1)
    def _extract():
        lane = lax.broadcasted_iota(jnp.int32, (1, 128), 1)
        rawv_ref[...] = jnp.zeros((1, 128), jnp.float32)
        rawi_ref[...] = jnp.zeros((1, 128), jnp.int32)

        bflat = (lax.broadcasted_iota(jnp.int32, (C, H // 8), 0) * (H // 8)
                 + lax.broadcasted_iota(jnp.int32, (C, H // 8), 1))
        ib = (lax.broadcasted_iota(jnp.int32, (8, W), 0) * W
              + lax.broadcasted_iota(jnp.int32, (8, W), 1))

        def body(i, carry):
            bm = bm_ref[...]
            m = jnp.max(bm)
            bsel = jnp.min(jnp.where(bm == m, bflat, jnp.int32(1 << 30)))
            tr0 = pl.multiple_of(bsel * 8, 8)
            blk = score_ref[pl.ds(tr0, 8), :]
            v = jnp.max(blk)
            pos = jnp.min(jnp.where(blk == v, ib, jnp.int32(1 << 30)))
            gidx = tr0 * W + pos
            blk_new = jnp.where(ib == pos, NEG, blk)
            score_ref[pl.ds(tr0, 8), :] = blk_new
            bm_ref[...] = jnp.where(bflat == bsel, jnp.max(blk_new), bm)
            rawv_ref[...] = jnp.where(lane == i, v, rawv_ref[...])
            rawi_ref[...] = jnp.where(lane == i, gidx, rawi_ref[...])
            row = tr0 + pos // W
            ph = row % H
            pw = pos % W
            pw0 = pl.multiple_of((pw // 128) * 128, 128)
            pltpu.make_async_copy(
                ks_ref.at[0, :, ph, pl.ds(pw0, 128)], stage_ref.at[:, i, :],
                dma_sem).start()
            rawp_ref[...] = jnp.where(lane == i, pw % 128, rawp_ref[...])
            return carry

        lax.fori_loop(0, K, body, 0)

        def drain(i, carry):
            pltpu.make_async_copy(
                ks_ref.at[0, :, 0, pl.ds(0, 128)], stage_ref.at[:, 0, :],
                dma_sem).wait()
            return carry

        lax.fori_loop(0, K, drain, 0)

        vals = rawv_ref[...]
        idx = rawi_ref[...]
        valid = lane < K
        keep = jnp.logical_and(vals > THRES, valid)
        svals_ref[...] = jnp.where(keep, vals, 0.0)
        scats_ref[...] = jnp.where(keep, idx // HW, 0)
        pwcol = jnp.transpose(rawp_ref[...])
        m8 = lax.broadcasted_iota(jnp.int32, (K, 128), 1) == pwcol[:K]
        st = stage_ref[...]
        sel = jnp.sum(jnp.where(m8[None], st, 0.0), axis=2)
        kout_ref[...] = jnp.transpose(sel)


def _detect(thing_map, kernel_space):
    return pl.pallas_call(
        _scores_topk_body,
        grid=(C,),
        in_specs=[
            pl.BlockSpec((1, 1, H, W), lambda c: (0, c, 0, 0)),
            pl.BlockSpec(memory_space=pltpu.MemorySpace.HBM),
        ],
        out_specs=[
            pl.BlockSpec((1, 128), lambda c: (0, 0)),
            pl.BlockSpec((1, 128), lambda c: (0, 0)),
            pl.BlockSpec((K, D), lambda c: (0, 0)),
        ],
        out_shape=[
            jax.ShapeDtypeStruct((1, 128), jnp.float32),
            jax.ShapeDtypeStruct((1, 128), jnp.int32),
            jax.ShapeDtypeStruct((K, D), jnp.float32),
        ],
        scratch_shapes=[
            pltpu.VMEM((C * H, W), jnp.float32),
            pltpu.VMEM((C, H // 8), jnp.float32),
            pltpu.VMEM((1, 128), jnp.float32),
            pltpu.VMEM((1, 128), jnp.int32),
            pltpu.VMEM((1, 128), jnp.int32),
            pltpu.VMEM((D, K, 128), jnp.float32),
            pltpu.SemaphoreType.DMA,
        ],
        compiler_params=pltpu.CompilerParams(
            dimension_semantics=("arbitrary",)),
    )(thing_map, kernel_space)


def kernel(thing_map, kernel_space):
    svals, scats, rows = _detect(thing_map, kernel_space)
    kernels = rows[None]
    scores = svals[:, :K]
    cats = scats[:, :K]
    return kernels, cats, scores
